# baseline (speedup 1.0000x reference)
import functools

import jax
import jax.numpy as jnp
from jax import lax
from jax.experimental import pallas as pl
from jax.experimental.pallas import tpu as pltpu
from jax.experimental.pallas import tpu_sc as plsc

N = 1024
NUM_CLASSES = 5532
QS = NUM_CLASSES * 2
FEAT = 256
L = 16

CPT = 352
NMAP = 16 * CPT // L
NBLK = CPT // L
NLBL = N // L
W = 64

TAIL0 = N
TAILC = 624
TAILR = 10040 - 16 * TAILC


def _sc_body(labels_hbm, qlabel_hbm,
             lbl_v, map_v, out_v, zbuf_v, bcnt_s):
    cid = lax.axis_index("c")
    sid = lax.axis_index("s")
    iota = lax.iota(jnp.int32, L)
    zf = jnp.zeros((L,), jnp.float32)
    zi = jnp.zeros((L,), jnp.int32)

    @pl.when(cid == 1)
    def _tail_zero():
        @pl.loop(0, TAILC // L, unroll=4)
        def _zb(k):
            zbuf_v[pl.ds(k * L, L)] = zf

        pltpu.sync_copy(zbuf_v.at[pl.ds(0, TAILC)],
                        qlabel_hbm.at[pl.ds(TAIL0 + sid * TAILC, TAILC)])

        @pl.when(sid == 0)
        def _tail_rem():
            pltpu.sync_copy(zbuf_v.at[pl.ds(0, TAILR)],
                            qlabel_hbm.at[pl.ds(TAIL0 + 16 * TAILC, TAILR)])

    @pl.when(cid == 0)
    def _compute():
        pltpu.sync_copy(labels_hbm, lbl_v)
        @pl.loop(0, NMAP, unroll=4)
        def _zero_map(k):
            map_v[pl.ds(k * L, L)] = zi

        for k in range(W // L):
            out_v[pl.ds(k * L, L)] = zf
        one = zi + 1

        @pl.loop(0, NLBL, unroll=4)
        def _mark(j):
            lbl = lbl_v[pl.ds(j * L, L)]
            plsc.store_scatter(map_v, [lbl], one)

        @pl.loop(0, 16)
        def _blocks(g):
            @pl.loop(0, NBLK, unroll=2, init_carry=zi)
            def _acc(k, acc):
                return acc + map_v[pl.ds((g * NBLK + k) * L, L)]

            bcnt_s[g] = jnp.sum(_acc)

        lo_s = W * sid
        iota_f = iota.astype(jnp.float32)

        @pl.loop(0, 16, init_carry=jnp.int32(0))
        def _scan(g, carry_s):
            bs = bcnt_s[g]

            @pl.when((carry_s < lo_s + W) & (carry_s + bs > lo_s))
            def _compact():
                lo = zi + lo_s

                @pl.loop(0, NBLK, unroll=2, init_carry=zi + carry_s)
                def _inner(k, carry):
                    p = map_v[pl.ds((g * NBLK + k) * L, L)]
                    pres = p > 0
                    cs = plsc.cumsum(p)
                    gr = (cs - p) + carry
                    li = gr - lo
                    m = pres & (li >= 0) & (li < W)
                    lic = jnp.minimum(jnp.maximum(li, 0), W - 1)
                    clsf = iota_f + ((g * NBLK + k) * L).astype(jnp.float32)
                    plsc.store_scatter(out_v, [lic], clsf, mask=m)
                    return carry + plsc.all_reduce_population_count(pres)

            return carry_s + bs

        pltpu.sync_copy(out_v, qlabel_hbm.at[pl.ds(W * sid, W)])


_sc_uniq = functools.partial(
    pl.kernel,
    mesh=plsc.VectorSubcoreMesh(core_axis_name="c", subcore_axis_name="s"),
    compiler_params=pltpu.CompilerParams(needs_layout_passes=False),
    out_type=[jax.ShapeDtypeStruct((QS,), jnp.float32)],
    scratch_types=[
        pltpu.VMEM((N,), jnp.int32),
        pltpu.VMEM((16 * CPT,), jnp.int32),
        pltpu.VMEM((W,), jnp.float32),
        pltpu.VMEM((TAILC,), jnp.float32),
        pltpu.SMEM((16,), jnp.int32),
    ],
)(_sc_body)


BR = 512
NFB = N // BR


def _queue_body(feat_ref, lrow_ref, out_ref):
    i = pl.program_id(0)

    @pl.when(i < NFB)
    def _copy():
        lrow = lrow_ref[...]
        lcol = jnp.transpose(lrow)
        eq = (lcol == lrow)
        ri = lax.broadcasted_iota(jnp.int32, (N, N), 0)
        ci = lax.broadcasted_iota(jnp.int32, (N, N), 1)
        dup = jnp.any(eq & (ci < ri), axis=1, keepdims=True)
        u = jnp.sum(1 - dup.astype(jnp.int32))
        rowi = lax.broadcasted_iota(jnp.int32, (BR, 1), 0) + i * BR
        mask = (rowi < u).astype(jnp.float32)
        out_ref[...] = feat_ref[...] * mask

    @pl.when(i >= NFB)
    def _zero():
        out_ref[...] = jnp.zeros_like(out_ref)


def kernel(features, pid_labels):
    (qlabel,) = _sc_uniq(pid_labels)
    queue = pl.pallas_call(
        _queue_body,
        grid=(pl.cdiv(QS, BR),),
        in_specs=[pl.BlockSpec((BR, FEAT),
                               lambda i: (jnp.minimum(i, NFB - 1), 0)),
                  pl.BlockSpec((1, N), lambda i: (0, 0))],
        out_specs=pl.BlockSpec((BR, FEAT), lambda i: (i, 0)),
        out_shape=jax.ShapeDtypeStruct((QS, FEAT), jnp.float32),
    )(features, pid_labels.reshape(1, N))
    return (queue, qlabel)


# speedup vs baseline: 1.0831x; 1.0831x over previous
import functools

import jax
import jax.numpy as jnp
from jax import lax
from jax.experimental import pallas as pl
from jax.experimental.pallas import tpu as pltpu
from jax.experimental.pallas import tpu_sc as plsc

N = 1024
NUM_CLASSES = 5532
QS = NUM_CLASSES * 2
FEAT = 256
L = 16

CPT = 352
NMAP = 16 * CPT // L
NBLK = CPT // L
NLBL = N // L
W = 64

TAIL0 = N
TAILC = 624
TAILR = 10040 - 16 * TAILC


def _sc_body(labels_hbm, qlabel_hbm,
             lbl_v, map_v, out_v, zbuf_v, bcnt_s):
    cid = lax.axis_index("c")
    sid = lax.axis_index("s")
    iota = lax.iota(jnp.int32, L)
    zf = jnp.zeros((L,), jnp.float32)
    zi = jnp.zeros((L,), jnp.int32)

    @pl.when(cid == 1)
    def _tail_zero():
        @pl.loop(0, TAILC // L, unroll=4)
        def _zb(k):
            zbuf_v[pl.ds(k * L, L)] = zf

        pltpu.sync_copy(zbuf_v.at[pl.ds(0, TAILC)],
                        qlabel_hbm.at[pl.ds(TAIL0 + sid * TAILC, TAILC)])

        @pl.when(sid == 0)
        def _tail_rem():
            pltpu.sync_copy(zbuf_v.at[pl.ds(0, TAILR)],
                            qlabel_hbm.at[pl.ds(TAIL0 + 16 * TAILC, TAILR)])

    @pl.when(cid == 0)
    def _compute():
        pltpu.sync_copy(labels_hbm, lbl_v)
        @pl.loop(0, NMAP, unroll=4)
        def _zero_map(k):
            map_v[pl.ds(k * L, L)] = zi

        for k in range(W // L):
            out_v[pl.ds(k * L, L)] = zf
        one = zi + 1

        @pl.loop(0, NLBL, unroll=4)
        def _mark(j):
            lbl = lbl_v[pl.ds(j * L, L)]
            plsc.store_scatter(map_v, [lbl], one)

        @pl.loop(0, 16)
        def _blocks(g):
            @pl.loop(0, NBLK, unroll=2, init_carry=zi)
            def _acc(k, acc):
                return acc + map_v[pl.ds((g * NBLK + k) * L, L)]

            bcnt_s[g] = jnp.sum(_acc)

        lo_s = W * sid
        iota_f = iota.astype(jnp.float32)

        @pl.loop(0, 16, init_carry=jnp.int32(0))
        def _scan(g, carry_s):
            bs = bcnt_s[g]

            @pl.when((carry_s < lo_s + W) & (carry_s + bs > lo_s))
            def _compact():
                lo = zi + lo_s

                @pl.loop(0, NBLK, unroll=2, init_carry=zi + carry_s)
                def _inner(k, carry):
                    p = map_v[pl.ds((g * NBLK + k) * L, L)]
                    pres = p > 0
                    cs = plsc.cumsum(p)
                    gr = (cs - p) + carry
                    li = gr - lo
                    m = pres & (li >= 0) & (li < W)
                    lic = jnp.minimum(jnp.maximum(li, 0), W - 1)
                    clsf = iota_f + ((g * NBLK + k) * L).astype(jnp.float32)
                    plsc.store_scatter(out_v, [lic], clsf, mask=m)
                    return carry + plsc.all_reduce_population_count(pres)

            return carry_s + bs

        pltpu.sync_copy(out_v, qlabel_hbm.at[pl.ds(W * sid, W)])


_sc_uniq = functools.partial(
    pl.kernel,
    mesh=plsc.VectorSubcoreMesh(core_axis_name="c", subcore_axis_name="s"),
    compiler_params=pltpu.CompilerParams(needs_layout_passes=False),
    out_type=[jax.ShapeDtypeStruct((QS,), jnp.float32)],
    scratch_types=[
        pltpu.VMEM((N,), jnp.int32),
        pltpu.VMEM((16 * CPT,), jnp.int32),
        pltpu.VMEM((W,), jnp.float32),
        pltpu.VMEM((TAILC,), jnp.float32),
        pltpu.SMEM((16,), jnp.int32),
    ],
)(_sc_body)


BR = 2048


def _queue_body(feat_ref, lrow_ref, out_ref):
    i = pl.program_id(0)

    @pl.when(i == 0)
    def _copy():
        lrow = lrow_ref[...]
        lcol = jnp.transpose(lrow)
        eq = (lcol == lrow)
        ri = lax.broadcasted_iota(jnp.int32, (N, N), 0)
        ci = lax.broadcasted_iota(jnp.int32, (N, N), 1)
        dup = jnp.any(eq & (ci < ri), axis=1, keepdims=True)
        u = jnp.sum(1 - dup.astype(jnp.int32))
        rowi = lax.broadcasted_iota(jnp.int32, (N, 1), 0)
        mask = (rowi < u).astype(jnp.float32)
        out_ref[pl.ds(0, N), :] = feat_ref[...] * mask
        out_ref[pl.ds(N, BR - N), :] = jnp.zeros((BR - N, FEAT), jnp.float32)

    @pl.when(i > 0)
    def _zero():
        out_ref[...] = jnp.zeros_like(out_ref)


def kernel(features, pid_labels):
    (qlabel,) = _sc_uniq(pid_labels)
    queue = pl.pallas_call(
        _queue_body,
        grid=(pl.cdiv(QS, BR),),
        in_specs=[pl.BlockSpec((N, FEAT), lambda i: (0, 0)),
                  pl.BlockSpec((1, N), lambda i: (0, 0))],
        out_specs=pl.BlockSpec((BR, FEAT), lambda i: (i, 0)),
        out_shape=jax.ShapeDtypeStruct((QS, FEAT), jnp.float32),
    )(features, pid_labels.reshape(1, N))
    return (queue, qlabel)


# speedup vs baseline: 1.1413x; 1.0537x over previous
import functools

import jax
import jax.numpy as jnp
from jax import lax
from jax.experimental import pallas as pl
from jax.experimental.pallas import tpu as pltpu
from jax.experimental.pallas import tpu_sc as plsc

N = 1024
NUM_CLASSES = 5532
QS = NUM_CLASSES * 2
FEAT = 256
L = 16

CPT = 352
NMAP = 16 * CPT // L
NBLK = CPT // L
NLBL = N // L
W = 64

TAIL0 = N
TAILC = 624
TAILR = 10040 - 16 * TAILC


def _sc_body(labels_hbm, qlabel_hbm,
             lbl_v, map_v, out_v, zbuf_v, bcnt_s, sem):
    cid = lax.axis_index("c")
    sid = lax.axis_index("s")
    iota = lax.iota(jnp.int32, L)
    zf = jnp.zeros((L,), jnp.float32)
    zi = jnp.zeros((L,), jnp.int32)

    @pl.when(cid == 0)
    def _compute():
        @pl.loop(0, TAILC // L, unroll=4)
        def _zb(k):
            zbuf_v[pl.ds(k * L, L)] = zf

        tail_cp = pltpu.async_copy(
            zbuf_v.at[pl.ds(0, TAILC)],
            qlabel_hbm.at[pl.ds(TAIL0 + sid * TAILC, TAILC)], sem)
        pltpu.sync_copy(labels_hbm, lbl_v)
        @pl.loop(0, NMAP, unroll=4)
        def _zero_map(k):
            map_v[pl.ds(k * L, L)] = zi

        for k in range(W // L):
            out_v[pl.ds(k * L, L)] = zf
        one = zi + 1

        @pl.loop(0, NLBL, unroll=4)
        def _mark(j):
            lbl = lbl_v[pl.ds(j * L, L)]
            plsc.store_scatter(map_v, [lbl], one)

        @pl.loop(0, 16)
        def _blocks(g):
            @pl.loop(0, NBLK, unroll=2, init_carry=zi)
            def _acc(k, acc):
                return acc + map_v[pl.ds((g * NBLK + k) * L, L)]

            bcnt_s[g] = jnp.sum(_acc)

        lo_s = W * sid
        iota_f = iota.astype(jnp.float32)

        @pl.loop(0, 16, init_carry=jnp.int32(0))
        def _scan(g, carry_s):
            bs = bcnt_s[g]

            @pl.when((carry_s < lo_s + W) & (carry_s + bs > lo_s))
            def _compact():
                lo = zi + lo_s

                @pl.loop(0, NBLK, unroll=2, init_carry=zi + carry_s)
                def _inner(k, carry):
                    p = map_v[pl.ds((g * NBLK + k) * L, L)]
                    pres = p > 0
                    cs = plsc.cumsum(p)
                    gr = (cs - p) + carry
                    li = gr - lo
                    m = pres & (li >= 0) & (li < W)
                    lic = jnp.minimum(jnp.maximum(li, 0), W - 1)
                    clsf = iota_f + ((g * NBLK + k) * L).astype(jnp.float32)
                    plsc.store_scatter(out_v, [lic], clsf, mask=m)
                    return carry + plsc.all_reduce_population_count(pres)

            return carry_s + bs

        pltpu.sync_copy(out_v, qlabel_hbm.at[pl.ds(W * sid, W)])

        @pl.when(sid == 0)
        def _tail_rem():
            pltpu.sync_copy(zbuf_v.at[pl.ds(0, TAILR)],
                            qlabel_hbm.at[pl.ds(TAIL0 + 16 * TAILC, TAILR)])

        tail_cp.wait()


_sc_uniq = functools.partial(
    pl.kernel,
    mesh=plsc.VectorSubcoreMesh(core_axis_name="c", subcore_axis_name="s",
                                num_cores=1),
    compiler_params=pltpu.CompilerParams(needs_layout_passes=False),
    out_type=[jax.ShapeDtypeStruct((QS,), jnp.float32)],
    scratch_types=[
        pltpu.VMEM((N,), jnp.int32),
        pltpu.VMEM((16 * CPT,), jnp.int32),
        pltpu.VMEM((W,), jnp.float32),
        pltpu.VMEM((TAILC,), jnp.float32),
        pltpu.SMEM((16,), jnp.int32),
        pltpu.SemaphoreType.DMA,
    ],
)(_sc_body)


BR = 2048


def _queue_body(feat_ref, lrow_ref, out_ref):
    i = pl.program_id(0)

    @pl.when(i == 0)
    def _copy():
        lrow = lrow_ref[...]
        lcol = jnp.transpose(lrow)
        eq = (lcol == lrow)
        ri = lax.broadcasted_iota(jnp.int32, (N, N), 0)
        ci = lax.broadcasted_iota(jnp.int32, (N, N), 1)
        dup = jnp.any(eq & (ci < ri), axis=1, keepdims=True)
        u = jnp.sum(1 - dup.astype(jnp.int32))
        rowi = lax.broadcasted_iota(jnp.int32, (N, 1), 0)
        mask = (rowi < u).astype(jnp.float32)
        out_ref[pl.ds(0, N), :] = feat_ref[...] * mask
        out_ref[pl.ds(N, BR - N), :] = jnp.zeros((BR - N, FEAT), jnp.float32)

    @pl.when(i > 0)
    def _zero():
        out_ref[...] = jnp.zeros_like(out_ref)


def kernel(features, pid_labels):
    (qlabel,) = _sc_uniq(pid_labels)
    queue = pl.pallas_call(
        _queue_body,
        grid=(pl.cdiv(QS, BR),),
        in_specs=[pl.BlockSpec((N, FEAT), lambda i: (0, 0)),
                  pl.BlockSpec((1, N), lambda i: (0, 0))],
        out_specs=pl.BlockSpec((BR, FEAT), lambda i: (i, 0)),
        out_shape=jax.ShapeDtypeStruct((QS, FEAT), jnp.float32),
    )(features, pid_labels.reshape(1, N))
    return (queue, qlabel)
